# baseline (speedup 1.0000x reference)
import jax
import jax.numpy as jnp
from jax.experimental import pallas as pl
from jax.experimental.pallas import tpu as pltpu

N = 4096
D = 256
TEMP = 0.5
LOG2E = 1.4426950408889634
LN2 = 0.6931471805599453
ROWSCALE = LOG2E / TEMP

BM = 4096
BN = 2048
H = N // BN
CH = 256
NCH = 8
CHUNK = N // NCH


def _main_body(a_hbm, b_hbm, o_ref, za, zb, zbt, araw, braw, sems):
    def a_copy(i):
        return pltpu.make_async_copy(
            a_hbm.at[pl.ds(i * CHUNK, CHUNK), :],
            araw.at[pl.ds(i * CHUNK, CHUNK), :],
            sems.at[i])

    def b_copy(i):
        return pltpu.make_async_copy(
            b_hbm.at[pl.ds(i * CHUNK, CHUNK), :],
            braw.at[pl.ds(i * CHUNK, CHUNK), :],
            sems.at[NCH + i])

    for i in range(NCH):
        b_copy(i).start()
    for i in range(NCH):
        a_copy(i).start()

    for i in range(NCH):
        b_copy(i).wait()
        for j in range(CHUNK // CH):
            r0 = i * CHUNK + j * CH
            x = braw[r0:r0 + CH, :]
            ss = jnp.sum(x * x, axis=1, keepdims=True)
            inv = 1.0 / jnp.maximum(jnp.sqrt(ss), 1e-12)
            z = x * inv
            zb[r0:r0 + CH, :] = z.astype(jnp.bfloat16)
            zbt[:, r0:r0 + CH] = jnp.swapaxes(z, 0, 1).astype(jnp.bfloat16)
    for i in range(NCH):
        a_copy(i).wait()
        for j in range(CHUNK // CH):
            r0 = i * CHUNK + j * CH
            x = araw[r0:r0 + CH, :]
            ss = jnp.sum(x * x, axis=1, keepdims=True)
            inv = ROWSCALE / jnp.maximum(jnp.sqrt(ss), 1e-12)
            za[r0:r0 + CH, :] = (x * inv).astype(jnp.bfloat16)

    rows = za[...]
    rowacc = jnp.zeros((BM, 1), jnp.float32)
    colacc = jnp.zeros((1, N), jnp.float32)
    colparts = []
    for hh in range(H):
        colsT = zbt[:, hh * BN:(hh + 1) * BN]
        s = jax.lax.dot_general(rows, colsT, (((1,), (0,)), ((), ())),
                                preferred_element_type=jnp.float32)
        e = jnp.exp2(s)
        rowacc = rowacc + jnp.sum(e, axis=1, keepdims=True)
        colparts.append(jnp.sum(e, axis=0, keepdims=True))
    colacc = jnp.concatenate(colparts, axis=1)

    posv = jnp.sum((rows * zb[...]).astype(jnp.float32), axis=1, keepdims=True)
    rowtot = jnp.sum(jnp.log(0.5 * rowacc) - posv * (2.0 * LN2))
    coltot = jnp.sum(jnp.log(0.5 * colacc))
    o_ref[0, 0] = (rowtot + coltot) / (2 * N)


def kernel(emb_i, emb_j):
    loss = pl.pallas_call(
        _main_body,
        in_specs=[
            pl.BlockSpec(memory_space=pl.ANY),
            pl.BlockSpec(memory_space=pl.ANY),
        ],
        out_specs=pl.BlockSpec(memory_space=pltpu.SMEM),
        out_shape=jax.ShapeDtypeStruct((1, 1), jnp.float32),
        scratch_shapes=[
            pltpu.VMEM((N, D), jnp.bfloat16),
            pltpu.VMEM((N, D), jnp.bfloat16),
            pltpu.VMEM((D, N), jnp.bfloat16),
            pltpu.VMEM((N, D), jnp.float32),
            pltpu.VMEM((N, D), jnp.float32),
            pltpu.SemaphoreType.DMA((2 * NCH,)),
        ],
        name="ntxent_fused",
    )(emb_i, emb_j)
    return loss[0, 0]


# speedup vs baseline: 1.0354x; 1.0354x over previous
import jax
import jax.numpy as jnp
from jax.experimental import pallas as pl
from jax.experimental.pallas import tpu as pltpu

N = 4096
D = 256
TEMP = 0.5
LOG2E = 1.4426950408889634
LN2 = 0.6931471805599453
ROWSCALE = LOG2E / TEMP

BM = 4096
BN = 2048
H = N // BN
CH = 256
NCH = 4
CHUNK = N // NCH


def _main_body(a_hbm, b_hbm, o_ref, za, zb, zbt, araw, braw, sems):
    def a_copy(i):
        return pltpu.make_async_copy(
            a_hbm.at[pl.ds(i * CHUNK, CHUNK), :],
            araw.at[pl.ds(i * CHUNK, CHUNK), :],
            sems.at[i])

    def b_copy(i):
        return pltpu.make_async_copy(
            b_hbm.at[pl.ds(i * CHUNK, CHUNK), :],
            braw.at[pl.ds(i * CHUNK, CHUNK), :],
            sems.at[NCH + i])

    for i in range(NCH):
        b_copy(i).start()
    for i in range(NCH):
        a_copy(i).start()

    for i in range(NCH):
        b_copy(i).wait()
        for j in range(CHUNK // CH):
            r0 = i * CHUNK + j * CH
            x = braw[r0:r0 + CH, :]
            ss = jnp.sum(x * x, axis=1, keepdims=True)
            inv = 1.0 / jnp.maximum(jnp.sqrt(ss), 1e-12)
            z = x * inv
            zb[r0:r0 + CH, :] = z.astype(jnp.bfloat16)
            zbt[:, r0:r0 + CH] = jnp.swapaxes(z, 0, 1).astype(jnp.bfloat16)
    for i in range(NCH):
        a_copy(i).wait()
        for j in range(CHUNK // CH):
            r0 = i * CHUNK + j * CH
            x = araw[r0:r0 + CH, :]
            ss = jnp.sum(x * x, axis=1, keepdims=True)
            inv = ROWSCALE / jnp.maximum(jnp.sqrt(ss), 1e-12)
            za[r0:r0 + CH, :] = (x * inv).astype(jnp.bfloat16)

    rows = za[...]
    rowacc = jnp.zeros((BM, 1), jnp.float32)
    colacc = jnp.zeros((1, N), jnp.float32)
    colparts = []
    for hh in range(H):
        colsT = zbt[:, hh * BN:(hh + 1) * BN]
        s = jax.lax.dot_general(rows, colsT, (((1,), (0,)), ((), ())),
                                preferred_element_type=jnp.float32)
        e = jnp.exp2(s)
        rowacc = rowacc + jnp.sum(e, axis=1, keepdims=True)
        colparts.append(jnp.sum(e, axis=0, keepdims=True))
    colacc = jnp.concatenate(colparts, axis=1)

    posv = jnp.sum((rows * zb[...]).astype(jnp.float32), axis=1, keepdims=True)
    rowtot = jnp.sum(jnp.log(0.5 * rowacc) - posv * (2.0 * LN2))
    coltot = jnp.sum(jnp.log(0.5 * colacc))
    o_ref[0, 0] = (rowtot + coltot) / (2 * N)


def kernel(emb_i, emb_j):
    loss = pl.pallas_call(
        _main_body,
        in_specs=[
            pl.BlockSpec(memory_space=pl.ANY),
            pl.BlockSpec(memory_space=pl.ANY),
        ],
        out_specs=pl.BlockSpec(memory_space=pltpu.SMEM),
        out_shape=jax.ShapeDtypeStruct((1, 1), jnp.float32),
        scratch_shapes=[
            pltpu.VMEM((N, D), jnp.bfloat16),
            pltpu.VMEM((N, D), jnp.bfloat16),
            pltpu.VMEM((D, N), jnp.bfloat16),
            pltpu.VMEM((N, D), jnp.float32),
            pltpu.VMEM((N, D), jnp.float32),
            pltpu.SemaphoreType.DMA((2 * NCH,)),
        ],
        name="ntxent_fused",
    )(emb_i, emb_j)
    return loss[0, 0]
